# baseline (speedup 1.0000x reference)
import jax
import jax.numpy as jnp
from jax import lax
from jax.experimental import pallas as pl
from jax.experimental.pallas import tpu as pltpu
from jax.experimental.pallas import tpu_sc as plsc

N = 10000
E = 320000
D = 128
NID = 1000
MLP_H = 256
C_OUT = 6

NC = 2
NS = 16
NW = NC * NS
CH = 128
NCHUNK = 80
EPAD = NW * NCHUNK * CH - E
GCH = 8
NGROUP = NCHUNK // GCH
NP = 10240
ROWS_PER_TILE = NP // NS

import functools


@functools.cache
def _mesh():
    return plsc.VectorSubcoreMesh(
        core_axis_name="c", subcore_axis_name="s",
        num_cores=NC, num_subcores=NS)


NSUPER = NCHUNK // 16


def _make_seg(with_deg):
    def body(*args):
        if with_deg:
            (h_hbm, edge_hbm, zeros_hbm, agg_out, deg_out,
             ies, ied, ios, iod, rba, rbb,
             semga, semgb, semsa, semsb, acc, hist) = args
        else:
            (h_hbm, edge_hbm, zeros_hbm, agg_out,
             ies, ied, ios, iod, rba, rbb,
             semga, semgb, semsa, semsb, acc) = args

        cid = lax.axis_index("c")
        sid = lax.axis_index("s")
        wid = sid * NC + cid
        r0 = sid * ROWS_PER_TILE
        base = wid * NCHUNK

        pltpu.sync_copy(zeros_hbm.at[pl.ds(r0, ROWS_PER_TILE)],
                        acc.at[pl.ds(r0, ROWS_PER_TILE)])
        if with_deg:
            zv = jnp.zeros((16,), jnp.float32)

            def z(i, c):
                hist[pl.ds(i * 16, 16)] = zv
                return c

            lax.fori_loop(0, NP // 16, z, 0)
        plsc.subcore_barrier()

        ones = jnp.ones((16,), jnp.float32)

        def stage(g, bs, bd):
            pltpu.sync_copy(edge_hbm.at[pl.ds(base + g * GCH, GCH)], bs)
            pltpu.sync_copy(
                edge_hbm.at[pl.ds(NW * NCHUNK + base + g * GCH, GCH)], bd)

        def start_g(rb, sem, bs, row):
            pltpu.async_copy(h_hbm.at[bs.at[row]], rb, sem)

        def wait_g(rb, sem):
            pltpu.make_async_copy(h_hbm.at[ies.at[0]], rb, sem).wait()

        def issue_s(rb, sem, bd, row):
            pltpu.async_copy(rb, acc.at[bd.at[row]], sem, add=True)

        def wait_s(sem):
            pltpu.make_async_copy(rba, acc.at[ied.at[0]], sem).wait()

        def hist_upd(bd, row):
            for k in range(8):
                plsc.addupdate_scatter(
                    hist, [bd[row, pl.ds(k * 16, 16)]], ones)

        def step(j, cur_rb, cur_semg, cur_sems, oth_rb, oth_semg, oth_sems,
                 bs_cur, srow, bd_prev, prow):
            @pl.when(j >= 2)
            def _():
                wait_s(cur_sems)

            start_g(cur_rb, cur_semg, bs_cur, srow)

            @pl.when(j >= 1)
            def _():
                wait_g(oth_rb, oth_semg)
                issue_s(oth_rb, oth_sems, bd_prev, prow)

            if with_deg:
                @pl.when(j >= 1)
                def _():
                    hist_upd(bd_prev, prow)

        bufs = ((rba, semga, semsa), (rbb, semgb, semsb))

        def super_body(sg, carry):
            stage(2 * sg, ies, ied)
            for local in range(16):
                j = 16 * sg + local
                if local == 8:
                    stage(2 * sg + 1, ios, iod)
                cur = bufs[local % 2]
                oth = bufs[1 - local % 2]
                bs_cur, srow = (ies, local) if local < 8 else (ios, local - 8)
                if local == 0:
                    bd_prev, prow = iod, 7
                elif local <= 8:
                    bd_prev, prow = ied, local - 1
                else:
                    bd_prev, prow = iod, local - 9
                step(j, *cur, *oth, bs_cur, srow, bd_prev, prow)
            return carry

        lax.fori_loop(0, NSUPER, super_body, 0)

        wait_g(rbb, semgb)
        issue_s(rbb, semsb, iod, 7)
        if with_deg:
            hist_upd(iod, 7)
        wait_s(semsa)
        wait_s(semsb)
        plsc.subcore_barrier()

        pltpu.sync_copy(acc.at[pl.ds(r0, ROWS_PER_TILE)],
                        agg_out.at[pl.ds(cid * NP + r0, ROWS_PER_TILE)])
        if with_deg:
            pltpu.sync_copy(hist, deg_out.at[pl.ds(wid * NP, NP)])

    out_type = [jax.ShapeDtypeStruct((NC * NP, D), jnp.float32)]
    if with_deg:
        out_type.append(jax.ShapeDtypeStruct((NW * NP,), jnp.float32))
    scratch = [
        pltpu.VMEM((GCH, CH), jnp.int32),
        pltpu.VMEM((GCH, CH), jnp.int32),
        pltpu.VMEM((GCH, CH), jnp.int32),
        pltpu.VMEM((GCH, CH), jnp.int32),
        pltpu.VMEM((CH, D), jnp.float32),
        pltpu.VMEM((CH, D), jnp.float32),
        pltpu.SemaphoreType.DMA,
        pltpu.SemaphoreType.DMA,
        pltpu.SemaphoreType.DMA,
        pltpu.SemaphoreType.DMA,
        pltpu.VMEM_SHARED((NP, D), jnp.float32),
    ]
    if with_deg:
        scratch.append(pltpu.VMEM((NP,), jnp.float32))
    return pl.kernel(
        body,
        out_type=tuple(out_type),
        mesh=_mesh(),
        compiler_params=pltpu.CompilerParams(needs_layout_passes=False),
        scratch_types=scratch,
    )


@functools.cache
def _seg_sum_deg():
    return _make_seg(True)


@functools.cache
def _seg_sum():
    return _make_seg(False)


BLK = 2048


def _id_mask(i, id_ref):
    rows = i * BLK + lax.broadcasted_iota(jnp.int32, (BLK, 1), 0)
    m = None
    for k in range(id_ref.shape[0]):
        eq = rows == id_ref[k, :][None, :]
        mk = jnp.any(eq, axis=1, keepdims=True)
        m = mk if m is None else (m | mk)
    return m


def _pre_body(x_ref, id_ref, ws_ref, wi_ref, b_ref, o_ref):
    i = pl.program_id(0)
    h = x_ref[...]
    st = jnp.dot(h, ws_ref[...], preferred_element_type=jnp.float32)
    it = jnp.dot(h, wi_ref[...], preferred_element_type=jnp.float32)
    o_ref[...] = jnp.where(_id_mask(i, id_ref), it, st) + b_ref[...]


def _agg_mean(agg_ref, deg_ref):
    agg = agg_ref[0] + agg_ref[1]
    deg = lax.dot_general(deg_ref[...], jnp.ones((NW, 1), jnp.float32),
                          (((0,), (0,)), ((), ())),
                          preferred_element_type=jnp.float32)
    return agg / jnp.maximum(deg, 1.0)


def _post_body(sel_ref, agg_ref, deg_ref, wn_ref, o_ref):
    nb = jnp.dot(_agg_mean(agg_ref, deg_ref), wn_ref[...],
                 preferred_element_type=jnp.float32)
    o_ref[...] = jnp.maximum(sel_ref[...] + nb, 0.0)


def _post_mlp_body(sel_ref, agg_ref, deg_ref, wn_ref, w1_ref, b1_ref,
                   w2_ref, b2_ref, o_ref):
    nb = jnp.dot(_agg_mean(agg_ref, deg_ref), wn_ref[...],
                 preferred_element_type=jnp.float32)
    h2 = jnp.maximum(sel_ref[...] + nb, 0.0)
    z = jnp.maximum(
        jnp.dot(h2, w1_ref[...], preferred_element_type=jnp.float32)
        + b1_ref[...], 0.0)
    o_ref[...] = jnp.dot(z, w2_ref[...],
                         preferred_element_type=jnp.float32) + b2_ref[...]


def _row_spec(width):
    return pl.BlockSpec((BLK, width), lambda i: (i, 0))


def _full_spec(shape):
    nd = len(shape)
    return pl.BlockSpec(shape, lambda i, _n=nd: (0,) * _n)


_pre = pl.pallas_call(
    _pre_body,
    grid=(NP // BLK,),
    in_specs=[
        _row_spec(D),
        _full_spec((8, 128)),
        _full_spec((D, D)),
        _full_spec((D, D)),
        _full_spec((1, D)),
    ],
    out_specs=_row_spec(D),
    out_shape=jax.ShapeDtypeStruct((NP, D), jnp.float32),
)

_POST_SPECS = [
    _row_spec(D),
    pl.BlockSpec((NC, BLK, D), lambda i: (0, i, 0)),
    pl.BlockSpec((NW, BLK), lambda i: (0, i)),
    _full_spec((D, D)),
]

_post = pl.pallas_call(
    _post_body,
    grid=(NP // BLK,),
    in_specs=_POST_SPECS,
    out_specs=_row_spec(D),
    out_shape=jax.ShapeDtypeStruct((NP, D), jnp.float32),
)

_post_mlp = pl.pallas_call(
    _post_mlp_body,
    grid=(NP // BLK,),
    in_specs=_POST_SPECS + [
        _full_spec((D, MLP_H)),
        _full_spec((1, MLP_H)),
        _full_spec((MLP_H, 128)),
        _full_spec((1, 128)),
    ],
    out_specs=_row_spec(128),
    out_shape=jax.ShapeDtypeStruct((NP, 128), jnp.float32),
)


def kernel(x, edge_index, id_index, extra, W_self_0, W_id_0, W_nb_0, b_0,
           W_self_1, W_id_1, W_nb_1, b_1, W_mlp1, b_mlp1, W_mlp2, b_mlp2):
    f32 = jnp.float32
    pad_src = (jnp.arange(EPAD, dtype=jnp.int32) * 37) % N
    pad_dst = N + (jnp.arange(EPAD, dtype=jnp.int32) % (NP - N))
    edges = jnp.concatenate(
        [edge_index, jnp.stack([pad_src, pad_dst])], axis=1)
    edges = edges.reshape(2 * NW * NCHUNK, CH)
    zeros_nd = jnp.zeros((NP, D), f32)
    x_pad = jnp.zeros((NP, D), f32).at[:N].set(x)
    idp = jnp.full((1024,), -1, jnp.int32).at[:NID].set(id_index).reshape(8, 128)
    W2p = jnp.zeros((MLP_H, 128), f32).at[:, :C_OUT].set(W_mlp2)
    b2p = jnp.zeros((1, 128), f32).at[0, :C_OUT].set(b_mlp2)

    sel1 = _pre(x_pad, idp, W_self_0, W_id_0, b_0.reshape(1, D))
    agg0, deghist = _seg_sum_deg()(x_pad, edges, zeros_nd)
    deg = deghist.reshape(NW, NP)
    h1 = _post(sel1, agg0.reshape(NC, NP, D), deg, W_nb_0)
    sel2 = _pre(h1, idp, W_self_1, W_id_1, b_1.reshape(1, D))
    (agg1,) = _seg_sum()(h1, edges, zeros_nd)
    out_pad = _post_mlp(sel2, agg1.reshape(NC, NP, D), deg, W_nb_1,
                        W_mlp1, b_mlp1.reshape(1, MLP_H), W2p, b2p)
    return out_pad[:N, :C_OUT]


# speedup vs baseline: 1.0193x; 1.0193x over previous
import jax
import jax.numpy as jnp
from jax import lax
from jax.experimental import pallas as pl
from jax.experimental.pallas import tpu as pltpu
from jax.experimental.pallas import tpu_sc as plsc

N = 10000
E = 320000
D = 128
NID = 1000
MLP_H = 256
C_OUT = 6

NC = 2
NS = 16
NW = NC * NS
CH = 125
EPW = E // NW
NCHUNK = EPW // CH
GCH = 8
NGROUP = NCHUNK // GCH
NP = 10240
ROWS_PER_TILE = NP // NS

import functools


@functools.cache
def _mesh():
    return plsc.VectorSubcoreMesh(
        core_axis_name="c", subcore_axis_name="s",
        num_cores=NC, num_subcores=NS)


NSUPER = NCHUNK // 16


def _make_seg(with_deg):
    def body(*args):
        if with_deg:
            (h_hbm, edge_hbm, zeros_hbm, agg_out, deg_out,
             ies, ied, ios, iod, rba, rbb,
             semga, semgb, semsa, semsb, acc, hist) = args
        else:
            (h_hbm, edge_hbm, zeros_hbm, agg_out,
             ies, ied, ios, iod, rba, rbb,
             semga, semgb, semsa, semsb, acc) = args

        cid = lax.axis_index("c")
        sid = lax.axis_index("s")
        wid = sid * NC + cid
        r0 = sid * ROWS_PER_TILE
        base = wid * NCHUNK

        pltpu.sync_copy(zeros_hbm.at[pl.ds(r0, ROWS_PER_TILE)],
                        acc.at[pl.ds(r0, ROWS_PER_TILE)])
        if with_deg:
            zv = jnp.zeros((16,), jnp.float32)

            def z(i, c):
                hist[pl.ds(i * 16, 16)] = zv
                return c

            lax.fori_loop(0, NP // 16, z, 0)
        plsc.subcore_barrier()

        ones = jnp.ones((16,), jnp.float32)
        tmask = jnp.arange(16, dtype=jnp.int32) >= 3

        def stage(g, bs, bd):
            pltpu.sync_copy(edge_hbm.at[pl.ds(base + g * GCH, GCH)], bs)
            pltpu.sync_copy(
                edge_hbm.at[pl.ds(NW * NCHUNK + base + g * GCH, GCH)], bd)

        def start_g(rb, sem, bs, row):
            pltpu.async_copy(h_hbm.at[bs.at[row]], rb, sem)

        def wait_g(rb, sem):
            pltpu.make_async_copy(h_hbm.at[ies.at[0]], rb, sem).wait()

        def issue_s(rb, sem, bd, row):
            pltpu.async_copy(rb, acc.at[bd.at[row]], sem, add=True)

        def wait_s(sem):
            pltpu.make_async_copy(rba, acc.at[ied.at[0]], sem).wait()

        def hist_upd(bd, row):
            for k in range(7):
                plsc.addupdate_scatter(
                    hist, [bd[row, pl.ds(k * 16, 16)]], ones)
            plsc.addupdate_scatter(
                hist, [bd[row, pl.ds(CH - 16, 16)]], ones, mask=tmask)

        def step(j, cur_rb, cur_semg, cur_sems, oth_rb, oth_semg, oth_sems,
                 bs_cur, srow, bd_prev, prow):
            @pl.when(j >= 2)
            def _():
                wait_s(cur_sems)

            start_g(cur_rb, cur_semg, bs_cur, srow)

            @pl.when(j >= 1)
            def _():
                wait_g(oth_rb, oth_semg)
                issue_s(oth_rb, oth_sems, bd_prev, prow)

            if with_deg:
                @pl.when(j >= 1)
                def _():
                    hist_upd(bd_prev, prow)

        bufs = ((rba, semga, semsa), (rbb, semgb, semsb))

        def super_body(sg, carry):
            stage(2 * sg, ies, ied)
            for local in range(16):
                j = 16 * sg + local
                if local == 8:
                    stage(2 * sg + 1, ios, iod)
                cur = bufs[local % 2]
                oth = bufs[1 - local % 2]
                bs_cur, srow = (ies, local) if local < 8 else (ios, local - 8)
                if local == 0:
                    bd_prev, prow = iod, 7
                elif local <= 8:
                    bd_prev, prow = ied, local - 1
                else:
                    bd_prev, prow = iod, local - 9
                step(j, *cur, *oth, bs_cur, srow, bd_prev, prow)
            return carry

        lax.fori_loop(0, NSUPER, super_body, 0)

        wait_g(rbb, semgb)
        issue_s(rbb, semsb, iod, 7)
        if with_deg:
            hist_upd(iod, 7)
        wait_s(semsa)
        wait_s(semsb)
        plsc.subcore_barrier()

        pltpu.sync_copy(acc.at[pl.ds(r0, ROWS_PER_TILE)],
                        agg_out.at[pl.ds(cid * NP + r0, ROWS_PER_TILE)])
        if with_deg:
            pltpu.sync_copy(hist, deg_out.at[pl.ds(wid * NP, NP)])

    out_type = [jax.ShapeDtypeStruct((NC * NP, D), jnp.float32)]
    if with_deg:
        out_type.append(jax.ShapeDtypeStruct((NW * NP,), jnp.float32))
    scratch = [
        pltpu.VMEM((GCH, CH), jnp.int32),
        pltpu.VMEM((GCH, CH), jnp.int32),
        pltpu.VMEM((GCH, CH), jnp.int32),
        pltpu.VMEM((GCH, CH), jnp.int32),
        pltpu.VMEM((CH, D), jnp.float32),
        pltpu.VMEM((CH, D), jnp.float32),
        pltpu.SemaphoreType.DMA,
        pltpu.SemaphoreType.DMA,
        pltpu.SemaphoreType.DMA,
        pltpu.SemaphoreType.DMA,
        pltpu.VMEM_SHARED((NP, D), jnp.float32),
    ]
    if with_deg:
        scratch.append(pltpu.VMEM((NP,), jnp.float32))
    return pl.kernel(
        body,
        out_type=tuple(out_type),
        mesh=_mesh(),
        compiler_params=pltpu.CompilerParams(needs_layout_passes=False),
        scratch_types=scratch,
    )


@functools.cache
def _seg_sum_deg():
    return _make_seg(True)


@functools.cache
def _seg_sum():
    return _make_seg(False)


BLK = 2048


def _id_mask(i, id_ref):
    rows = i * BLK + lax.broadcasted_iota(jnp.int32, (BLK, 1), 0)
    m = None
    for k in range(id_ref.shape[0]):
        eq = rows == id_ref[k, :][None, :]
        mk = jnp.any(eq, axis=1, keepdims=True)
        m = mk if m is None else (m | mk)
    return m


def _pre_body(x_ref, id_ref, ws_ref, wi_ref, b_ref, o_ref):
    i = pl.program_id(0)
    h = x_ref[...]
    st = jnp.dot(h, ws_ref[...], preferred_element_type=jnp.float32)
    it = jnp.dot(h, wi_ref[...], preferred_element_type=jnp.float32)
    o_ref[...] = jnp.where(_id_mask(i, id_ref), it, st) + b_ref[...]


def _agg_mean(agg_ref, deg_ref):
    agg = agg_ref[0] + agg_ref[1]
    deg = lax.dot_general(deg_ref[...], jnp.ones((NW, 1), jnp.float32),
                          (((0,), (0,)), ((), ())),
                          preferred_element_type=jnp.float32)
    return agg / jnp.maximum(deg, 1.0)


def _post_body(sel_ref, agg_ref, deg_ref, wn_ref, o_ref):
    nb = jnp.dot(_agg_mean(agg_ref, deg_ref), wn_ref[...],
                 preferred_element_type=jnp.float32)
    o_ref[...] = jnp.maximum(sel_ref[...] + nb, 0.0)


def _post_mlp_body(sel_ref, agg_ref, deg_ref, wn_ref, w1_ref, b1_ref,
                   w2_ref, b2_ref, o_ref):
    nb = jnp.dot(_agg_mean(agg_ref, deg_ref), wn_ref[...],
                 preferred_element_type=jnp.float32)
    h2 = jnp.maximum(sel_ref[...] + nb, 0.0)
    z = jnp.maximum(
        jnp.dot(h2, w1_ref[...], preferred_element_type=jnp.float32)
        + b1_ref[...], 0.0)
    o_ref[...] = jnp.dot(z, w2_ref[...],
                         preferred_element_type=jnp.float32) + b2_ref[...]


def _row_spec(width):
    return pl.BlockSpec((BLK, width), lambda i: (i, 0))


def _full_spec(shape):
    nd = len(shape)
    return pl.BlockSpec(shape, lambda i, _n=nd: (0,) * _n)


_pre = pl.pallas_call(
    _pre_body,
    grid=(NP // BLK,),
    in_specs=[
        _row_spec(D),
        _full_spec((8, 128)),
        _full_spec((D, D)),
        _full_spec((D, D)),
        _full_spec((1, D)),
    ],
    out_specs=_row_spec(D),
    out_shape=jax.ShapeDtypeStruct((NP, D), jnp.float32),
)

_POST_SPECS = [
    _row_spec(D),
    pl.BlockSpec((NC, BLK, D), lambda i: (0, i, 0)),
    pl.BlockSpec((NW, BLK), lambda i: (0, i)),
    _full_spec((D, D)),
]

_post = pl.pallas_call(
    _post_body,
    grid=(NP // BLK,),
    in_specs=_POST_SPECS,
    out_specs=_row_spec(D),
    out_shape=jax.ShapeDtypeStruct((NP, D), jnp.float32),
)

_post_mlp = pl.pallas_call(
    _post_mlp_body,
    grid=(NP // BLK,),
    in_specs=_POST_SPECS + [
        _full_spec((D, MLP_H)),
        _full_spec((1, MLP_H)),
        _full_spec((MLP_H, 128)),
        _full_spec((1, 128)),
    ],
    out_specs=_row_spec(128),
    out_shape=jax.ShapeDtypeStruct((NP, 128), jnp.float32),
)


def kernel(x, edge_index, id_index, extra, W_self_0, W_id_0, W_nb_0, b_0,
           W_self_1, W_id_1, W_nb_1, b_1, W_mlp1, b_mlp1, W_mlp2, b_mlp2):
    f32 = jnp.float32
    edges = edge_index.reshape(2 * NW * NCHUNK, CH)
    zeros_nd = jnp.zeros((NP, D), f32)
    x_pad = jnp.zeros((NP, D), f32).at[:N].set(x)
    idp = jnp.full((1024,), -1, jnp.int32).at[:NID].set(id_index).reshape(8, 128)
    W2p = jnp.zeros((MLP_H, 128), f32).at[:, :C_OUT].set(W_mlp2)
    b2p = jnp.zeros((1, 128), f32).at[0, :C_OUT].set(b_mlp2)

    sel1 = _pre(x_pad, idp, W_self_0, W_id_0, b_0.reshape(1, D))
    agg0, deghist = _seg_sum_deg()(x_pad, edges, zeros_nd)
    deg = deghist.reshape(NW, NP)
    h1 = _post(sel1, agg0.reshape(NC, NP, D), deg, W_nb_0)
    sel2 = _pre(h1, idp, W_self_1, W_id_1, b_1.reshape(1, D))
    (agg1,) = _seg_sum()(h1, edges, zeros_nd)
    out_pad = _post_mlp(sel2, agg1.reshape(NC, NP, D), deg, W_nb_1,
                        W_mlp1, b_mlp1.reshape(1, MLP_H), W2p, b2p)
    return out_pad[:N, :C_OUT]
